# Initial kernel scaffold; baseline (speedup 1.0000x reference)
import jax
import jax.numpy as jnp
from jax.experimental import pallas as pl


def kernel(x_idx, edge_index, edge_attr_idx, batch_index, node_emb, edge_emb, edge_enc_W, edge_enc_b, pre_W, pre_b, post_W, post_b, lin_W, lin_b, bn_g, bn_b, mlp_W1, mlp_b1, mlp_W2, mlp_b2, mlp_W3, mlp_b3):
    raise NotImplementedError("write your pallas kernel here")


# speedup vs baseline: 10.8897x; 10.8897x over previous
import functools
from functools import partial

import jax
import jax.numpy as jnp
from jax import lax
from jax.experimental import pallas as pl
from jax.experimental.pallas import tpu as pltpu

N = 10000
E = 160000
L = 2
TOWERS = 5
F_IN = 75
F_OUT = 15
N_GRAPHS = 64

NP = 10240
FP = 384
FH = 128
BLK = 1280
NB = NP // BLK
BIG = 1e30


def _proj_body(h_ref, wi_ref, wj_ref, bf_ref, emb_ref, w2_ref, bt_ref,
               c_ref, wn_ref, t_ref):
    h = h_ref[...]
    c_ref[...] = jnp.dot(h, wi_ref[...], preferred_element_type=jnp.float32) + bf_ref[...]
    wn_ref[...] = jnp.dot(h, wj_ref[...], preferred_element_type=jnp.float32)
    t_ref[...] = jnp.dot(emb_ref[...], w2_ref[...], preferred_element_type=jnp.float32) + bt_ref[...]


def _proj(h, wi, wj, bf, emb_p, w2, bt):
    return pl.pallas_call(
        _proj_body,
        grid=(NB,),
        in_specs=[
            pl.BlockSpec((BLK, FH), lambda i: (i, 0)),
            pl.BlockSpec((FH, FP), lambda i: (0, 0)),
            pl.BlockSpec((FH, FP), lambda i: (0, 0)),
            pl.BlockSpec((1, FP), lambda i: (0, 0)),
            pl.BlockSpec((112, 64), lambda i: (0, 0)),
            pl.BlockSpec((64, FP), lambda i: (0, 0)),
            pl.BlockSpec((1, FP), lambda i: (0, 0)),
        ],
        out_specs=[
            pl.BlockSpec((BLK, FP), lambda i: (i, 0)),
            pl.BlockSpec((BLK, FP), lambda i: (i, 0)),
            pl.BlockSpec((112, FP), lambda i: (0, 0)),
        ],
        out_shape=[
            jax.ShapeDtypeStruct((NP, FP), jnp.float32),
            jax.ShapeDtypeStruct((NP, FP), jnp.float32),
            jax.ShapeDtypeStruct((112, FP), jnp.float32),
        ],
    )(h, wi, wj, bf, emb_p, w2, bt)


def _combine_body(stats_ref, c_ref, h_ref, cnt_ref, cntf_ref,
                  px_ref, ga_ref, gm_ref, gt_ref, bfin_ref,
                  y_ref, psum_ref, psq_ref):
    i = pl.program_id(0)
    cnt_full = cntf_ref[...]
    rows_full = lax.broadcasted_iota(jnp.int32, (NP, 1), 0)
    valid_full = rows_full < N
    avg_log = jnp.sum(jnp.where(valid_full, jnp.log(cnt_full + 1.0), 0.0)) / float(N)

    cnt = cnt_ref[...]
    deg = jnp.maximum(cnt, 1.0)
    logdeg = jnp.log(deg + 1.0)
    sa = logdeg / avg_log
    sb = avg_log / logdeg
    has = cnt > 0.0

    st = stats_ref[...]
    s_sum = st[:, 0:FP]
    s_sq = st[:, FP:2 * FP]
    s_mn = st[:, 2 * FP:3 * FP]
    s_mx = st[:, 3 * FP:4 * FP]
    c = c_ref[...]
    mean_w = s_sum / deg
    mean = jnp.where(has, c + mean_w, 0.0)
    mn = jnp.where(has, c + s_mn, 0.0)
    mx = jnp.where(has, c + s_mx, 0.0)
    var = jax.nn.relu(s_sq / deg - mean_w * mean_w)
    std = jnp.where(has, jnp.sqrt(var + 1e-5), jnp.sqrt(1e-5))
    s_cat = jnp.concatenate([mean, mn, mx, std], axis=-1)

    h = h_ref[...]
    ya = jnp.dot(s_cat, ga_ref[...], preferred_element_type=jnp.float32)
    ym = jnp.dot(s_cat, gm_ref[...], preferred_element_type=jnp.float32)
    yt = jnp.dot(s_cat, gt_ref[...], preferred_element_type=jnp.float32)
    y = (jnp.dot(h, px_ref[...], preferred_element_type=jnp.float32)
         + ya + sa * ym + sb * yt + bfin_ref[...])
    y_ref[...] = y

    grow = lax.broadcasted_iota(jnp.int32, (BLK, 1), 0) + BLK * i
    vmask = grow < N
    ym_ = jnp.where(vmask, y, 0.0)
    psum_ref[...] = jnp.sum(ym_, axis=0).reshape(1, 1, FH)
    psq_ref[...] = jnp.sum(ym_ * ym_, axis=0).reshape(1, 1, FH)


def _combine(stats, c, h, cnt, px, ga, gm, gt, bfin):
    return pl.pallas_call(
        _combine_body,
        grid=(NB,),
        in_specs=[
            pl.BlockSpec((BLK, 4 * FP), lambda i: (i, 0)),
            pl.BlockSpec((BLK, FP), lambda i: (i, 0)),
            pl.BlockSpec((BLK, FH), lambda i: (i, 0)),
            pl.BlockSpec((BLK, 1), lambda i: (i, 0)),
            pl.BlockSpec((NP, 1), lambda i: (0, 0)),
            pl.BlockSpec((FH, FH), lambda i: (0, 0)),
            pl.BlockSpec((4 * FP, FH), lambda i: (0, 0)),
            pl.BlockSpec((4 * FP, FH), lambda i: (0, 0)),
            pl.BlockSpec((4 * FP, FH), lambda i: (0, 0)),
            pl.BlockSpec((1, FH), lambda i: (0, 0)),
        ],
        out_specs=[
            pl.BlockSpec((BLK, FH), lambda i: (i, 0)),
            pl.BlockSpec((1, 1, FH), lambda i: (i, 0, 0)),
            pl.BlockSpec((1, 1, FH), lambda i: (i, 0, 0)),
        ],
        out_shape=[
            jax.ShapeDtypeStruct((NP, FH), jnp.float32),
            jax.ShapeDtypeStruct((NB, 1, FH), jnp.float32),
            jax.ShapeDtypeStruct((NB, 1, FH), jnp.float32),
        ],
    )(stats, c, h, cnt, cnt, px, ga, gm, gt, bfin)


def _bn_body(y_ref, psum_ref, psq_ref, g_ref, b_ref, h_ref):
    i = pl.program_id(0)
    mu = jnp.sum(psum_ref[...], axis=0) / float(N)
    var = jnp.sum(psq_ref[...], axis=0) / float(N) - mu * mu
    inv = lax.rsqrt(var + 1e-5)
    y = (y_ref[...] - mu) * inv * g_ref[...] + b_ref[...]
    grow = lax.broadcasted_iota(jnp.int32, (BLK, 1), 0) + BLK * i
    h_ref[...] = jnp.where(grow < N, jax.nn.relu(y), 0.0)


def _bn(y, psum, psq, g, b):
    return pl.pallas_call(
        _bn_body,
        grid=(NB,),
        in_specs=[
            pl.BlockSpec((BLK, FH), lambda i: (i, 0)),
            pl.BlockSpec((NB, 1, FH), lambda i: (0, 0, 0)),
            pl.BlockSpec((NB, 1, FH), lambda i: (0, 0, 0)),
            pl.BlockSpec((1, FH), lambda i: (0, 0)),
            pl.BlockSpec((1, FH), lambda i: (0, 0)),
        ],
        out_specs=pl.BlockSpec((BLK, FH), lambda i: (i, 0)),
        out_shape=jax.ShapeDtypeStruct((NP, FH), jnp.float32),
    )(y, psum, psq, g, b)


def _pool_body(h_ref, bi_ref, w1_ref, b1_ref, w2_ref, b2_ref, w3_ref, b3_ref,
               acc_ref, out_ref):
    i = pl.program_id(0)

    @pl.when(i == 0)
    def _():
        acc_ref[...] = jnp.zeros_like(acc_ref)

    bi = bi_ref[...].reshape(1, BLK)
    gids = lax.broadcasted_iota(jnp.int32, (N_GRAPHS, BLK), 0)
    onehot_t = (gids == bi).astype(jnp.float32)
    acc_ref[...] += jnp.dot(onehot_t, h_ref[...], preferred_element_type=jnp.float32)

    @pl.when(i == NB - 1)
    def _():
        pooled = acc_ref[...]
        z = jax.nn.relu(jnp.dot(pooled, w1_ref[...], preferred_element_type=jnp.float32) + b1_ref[...])
        z = jax.nn.relu(jnp.dot(z, w2_ref[...], preferred_element_type=jnp.float32) + b2_ref[...])
        out_ref[...] = jnp.dot(z, w3_ref[...], preferred_element_type=jnp.float32) + b3_ref[...]


def _pool_mlp(h, bi, w1, b1, w2, b2, w3, b3):
    acc, out = pl.pallas_call(
        _pool_body,
        grid=(NB,),
        in_specs=[
            pl.BlockSpec((BLK, FH), lambda i: (i, 0)),
            pl.BlockSpec((BLK, 1), lambda i: (i, 0)),
            pl.BlockSpec((FH, 64), lambda i: (0, 0)),
            pl.BlockSpec((1, 64), lambda i: (0, 0)),
            pl.BlockSpec((64, 32), lambda i: (0, 0)),
            pl.BlockSpec((1, 32), lambda i: (0, 0)),
            pl.BlockSpec((32, 128), lambda i: (0, 0)),
            pl.BlockSpec((1, 128), lambda i: (0, 0)),
        ],
        out_specs=[
            pl.BlockSpec((N_GRAPHS, FH), lambda i: (0, 0)),
            pl.BlockSpec((N_GRAPHS, 128), lambda i: (0, 0)),
        ],
        out_shape=[
            jax.ShapeDtypeStruct((N_GRAPHS, FH), jnp.float32),
            jax.ShapeDtypeStruct((N_GRAPHS, 128), jnp.float32),
        ],
    )(h, bi, w1, b1, w2, b2, w3, b3)
    return out


def _prep_layer_weights(l, pre_W, pre_b, post_W, post_b, lin_W, lin_b,
                        edge_enc_W, edge_enc_b):
    F3 = TOWERS * F_IN
    Wi = pre_W[l][:, 0:F_IN, :].transpose(1, 0, 2).reshape(F_IN, F3)
    Wj = pre_W[l][:, F_IN:2 * F_IN, :].transpose(1, 0, 2).reshape(F_IN, F3)
    We = pre_W[l][:, 2 * F_IN:, :].transpose(1, 0, 2).reshape(F_IN, F3)
    b_flat = pre_b[l].reshape(F3)

    wi_p = jnp.zeros((FH, FP)).at[:F_IN, :F3].set(Wi)
    wj_p = jnp.zeros((FH, FP)).at[:F_IN, :F3].set(Wj)
    bf_p = jnp.zeros((1, FP)).at[0, :F3].set(b_flat)

    W2 = edge_enc_W[l] @ We
    w2_p = jnp.zeros((64, FP)).at[:50, :F3].set(W2)
    bt = edge_enc_b[l] @ We
    bt_p = jnp.zeros((1, FP)).at[0, :F3].set(bt)

    pw = post_W[l]
    lw = lin_W[l]

    def blockdiag(P):
        z = jnp.zeros((F3, TOWERS * F_OUT))
        for t in range(TOWERS):
            z = z.at[t * F_IN:(t + 1) * F_IN, t * F_OUT:(t + 1) * F_OUT].set(P[t])
        return z

    def gmat(row0):
        gs = []
        for s in range(4):
            bd = blockdiag(pw[:, row0 + s * F_IN: row0 + (s + 1) * F_IN, :]) @ lw
            gs.append(jnp.zeros((FP, FH)).at[:F3, :F_IN].set(bd))
        return jnp.concatenate(gs, axis=0)

    PX = jnp.zeros((F_IN, TOWERS * F_OUT))
    for t in range(TOWERS):
        PX = PX.at[:, t * F_OUT:(t + 1) * F_OUT].set(pw[t, 0:F_IN, :])
    PX = PX @ lw
    px_p = jnp.zeros((FH, FH)).at[:F_IN, :F_IN].set(PX)

    ga = gmat(F_IN)
    gm = gmat(F_IN + 300)
    gt = gmat(F_IN + 600)

    bvec = post_b[l].reshape(TOWERS * F_OUT) @ lw + lin_b[l]
    bfin = jnp.zeros((1, FH)).at[0, :F_IN].set(bvec)
    return wi_p, wj_p, bf_p, w2_p, bt_p, px_p, ga, gm, gt, bfin


def kernel(x_idx, edge_index, edge_attr_idx, batch_index, node_emb, edge_emb,
           edge_enc_W, edge_enc_b, pre_W, pre_b, post_W, post_b, lin_W, lin_b,
           bn_g, bn_b, mlp_W1, mlp_b1, mlp_W2, mlp_b2, mlp_W3, mlp_b3):
    src = edge_index[0]
    dst = edge_index[1]

    order = jnp.argsort(dst)
    src_s = src[order]
    ea_s = edge_attr_idx[order]
    dst_s = dst[order]
    offsets = jnp.searchsorted(dst_s, jnp.arange(NP + 1, dtype=jnp.int32)).astype(jnp.int32)
    cnt = (offsets[1:] - offsets[:-1]).astype(jnp.float32).reshape(NP, 1)

    h = jnp.zeros((NP, FH), jnp.float32).at[:N, :F_IN].set(node_emb[x_idx])
    bi = jnp.full((NP, 1), -1, jnp.int32).at[:N, 0].set(batch_index.astype(jnp.int32))
    emb_p = jnp.zeros((112, 64), jnp.float32).at[:100, :50].set(edge_emb)

    for l in range(L):
        wi_p, wj_p, bf_p, w2_p, bt_p, px_p, ga, gm, gt, bfin = _prep_layer_weights(
            l, pre_W, pre_b, post_W, post_b, lin_W, lin_b, edge_enc_W, edge_enc_b)
        c_arr, wn_arr, t_arr = _proj(h, wi_p, wj_p, bf_p, emb_p, w2_p, bt_p)

        w_e = wn_arr[src] + t_arr[edge_attr_idx]
        s_sum = jax.ops.segment_sum(w_e, dst, num_segments=NP)
        s_sq = jax.ops.segment_sum(w_e * w_e, dst, num_segments=NP)
        s_mn = jax.ops.segment_min(w_e, dst, num_segments=NP)
        s_mx = jax.ops.segment_max(w_e, dst, num_segments=NP)
        s_mn = jnp.clip(s_mn, -BIG, BIG)
        s_mx = jnp.clip(s_mx, -BIG, BIG)
        stats = jnp.concatenate([s_sum, s_sq, s_mn, s_mx], axis=-1)

        y, psum, psq = _combine(stats, c_arr, h, cnt, px_p, ga, gm, gt, bfin)
        g_p = jnp.zeros((1, FH)).at[0, :F_IN].set(bn_g[l])
        b_p = jnp.zeros((1, FH)).at[0, :F_IN].set(bn_b[l])
        h = _bn(y, psum, psq, g_p, b_p)

    w1_p = jnp.zeros((FH, 64)).at[:F_IN, :50].set(mlp_W1)
    b1_p = jnp.zeros((1, 64)).at[0, :50].set(mlp_b1)
    w2_p = jnp.zeros((64, 32)).at[:50, :25].set(mlp_W2)
    b2_p = jnp.zeros((1, 32)).at[0, :25].set(mlp_b2)
    w3_p = jnp.zeros((32, 128)).at[:25, :1].set(mlp_W3)
    b3_p = jnp.zeros((1, 128)).at[0, :1].set(mlp_b3)
    out = _pool_mlp(h, bi, w1_p, b1_p, w2_p, b2_p, w3_p, b3_p)
    return out[:, :1]


# speedup vs baseline: 14.2975x; 1.3129x over previous
import functools

import jax
import jax.numpy as jnp
from jax import lax
from jax.experimental import pallas as pl
from jax.experimental.pallas import tpu as pltpu
from jax.experimental.pallas import tpu_sc as plsc

N = 10000
E = 160000
L = 2
TOWERS = 5
F_IN = 75
F_OUT = 15
N_GRAPHS = 64

NP = 10240
FP = 384
FH = 128
BLK = 1280
NB = NP // BLK
BIG = 1e30
BF = jnp.bfloat16
F3 = TOWERS * F_IN


SC_NC = 2
SC_NS = 16
SC_NW = SC_NC * SC_NS
NPW = NP // SC_NW
NBLK = NPW // 16
CH = 64
KV = FP // 16
EPAD = E + 2 * CH


def _sload(ref, i):
    return ref[pl.ds(i, 16)][0]


def _sc_stats_body(wn_hbm, t_hbm, srcs_hbm, eas_hbm, dsts_hbm, offb_hbm,
                   out_hbm, srcbuf, eabuf, dstv, offv, wrows, trows, stage,
                   sem1, sem2):
    wid = lax.axis_index("s") * SC_NC + lax.axis_index("c")
    n0 = wid * NPW

    pltpu.sync_copy(offb_hbm.at[pl.ds(wid * 32, 32)], offv.at[pl.ds(0, 32)])

    def block_body(blk, _):
        sl = lax.rem(blk, 2)
        blk_n0 = n0 + blk * 16
        e0b = _sload(offv, blk)
        e1b = _sload(offv, blk + 1)
        c0 = pl.multiple_of(e0b & ~7, 8)
        nch = lax.div(e1b - c0 + (CH - 1), CH)

        zed = jnp.zeros((16,), jnp.float32)
        big = jnp.full((16,), BIG, jnp.float32)

        def init_body(s, _):
            for k in range(KV):
                d = pl.ds(k * 16, 16)
                stage[sl, s, 0, d] = zed
                stage[sl, s, 1, d] = zed
                stage[sl, s, 2, d] = big
                stage[sl, s, 3, d] = -big
            return 0

        lax.fori_loop(0, 16, init_body, 0)

        def chunk_body(ci, _):
            ce = pl.multiple_of(c0 + ci * CH, 8)
            pltpu.sync_copy(srcs_hbm.at[pl.ds(ce, CH)], srcbuf)
            pltpu.sync_copy(eas_hbm.at[pl.ds(ce, CH)], eabuf)
            pltpu.sync_copy(dsts_hbm.at[pl.ds(ce, CH)], dstv.at[pl.ds(0, CH)])
            g1 = pltpu.make_async_copy(wn_hbm.at[srcbuf], wrows, sem1)
            g2 = pltpu.make_async_copy(t_hbm.at[eabuf], trows, sem2)
            g1.start()
            g2.start()
            g1.wait()
            g2.wait()

            def edge_body(j, _):
                slot = _sload(dstv, j) - blk_n0

                @pl.when(jnp.logical_and(slot >= 0, slot < 16))
                def _():
                    for k in range(KV):
                        d = pl.ds(k * 16, 16)
                        w = wrows[j, d] + trows[j, d]
                        stage[sl, slot, 0, d] += w
                        stage[sl, slot, 1, d] += w * w
                        stage[sl, slot, 2, d] = jnp.minimum(stage[sl, slot, 2, d], w)
                        stage[sl, slot, 3, d] = jnp.maximum(stage[sl, slot, 3, d], w)

                return 0

            lax.fori_loop(0, CH, edge_body, 0)
            return 0

        lax.fori_loop(0, nch, chunk_body, 0)
        pltpu.sync_copy(stage.at[sl], out_hbm.at[pl.ds(blk_n0, 16)])
        return 0

    lax.fori_loop(0, NBLK, block_body, 0)


def _sc_stats(wn, t, srcs, eas, dsts, offb):
    f = functools.partial(
        pl.kernel,
        mesh=plsc.VectorSubcoreMesh(core_axis_name="c", subcore_axis_name="s"),
        out_type=jax.ShapeDtypeStruct((NP, 4, FP), jnp.float32),
        scratch_types=[
            pltpu.VMEM((CH,), jnp.int32),
            pltpu.VMEM((CH,), jnp.int32),
            pltpu.VMEM((CH + 16,), jnp.int32),
            pltpu.VMEM((48,), jnp.int32),
            pltpu.VMEM((CH, FP), jnp.float32),
            pltpu.VMEM((CH, FP), jnp.float32),
            pltpu.VMEM((2, 16, 4, FP), jnp.float32),
            pltpu.SemaphoreType.DMA,
            pltpu.SemaphoreType.DMA,
        ],
    )(_sc_stats_body)
    return f(wn, t, srcs, eas, dsts, offb)


def _proj_body(h_ref, wi_ref, wj_ref, bf_ref, emb_ref, encw_ref, encb_ref,
               we_ref, c_ref, wn_ref, t_ref):
    hb = h_ref[...].astype(BF)
    c_ref[...] = jnp.dot(hb, wi_ref[...].astype(BF),
                         preferred_element_type=jnp.float32) + bf_ref[...]
    wn_ref[...] = jnp.dot(hb, wj_ref[...].astype(BF),
                          preferred_element_type=jnp.float32)
    ea = jnp.dot(emb_ref[...].astype(BF), encw_ref[...].astype(BF),
                 preferred_element_type=jnp.float32) + encb_ref[...]
    t_ref[...] = jnp.dot(ea.astype(BF), we_ref[...].astype(BF),
                         preferred_element_type=jnp.float32)


def _proj(h, wi, wj, bf, emb_p, encw, encb, we):
    return pl.pallas_call(
        _proj_body,
        grid=(NB,),
        in_specs=[
            pl.BlockSpec((BLK, FH), lambda i: (i, 0)),
            pl.BlockSpec((FH, FP), lambda i: (0, 0)),
            pl.BlockSpec((FH, FP), lambda i: (0, 0)),
            pl.BlockSpec((1, FP), lambda i: (0, 0)),
            pl.BlockSpec((112, 64), lambda i: (0, 0)),
            pl.BlockSpec((64, FH), lambda i: (0, 0)),
            pl.BlockSpec((1, FH), lambda i: (0, 0)),
            pl.BlockSpec((FH, FP), lambda i: (0, 0)),
        ],
        out_specs=[
            pl.BlockSpec((BLK, FP), lambda i: (i, 0)),
            pl.BlockSpec((BLK, FP), lambda i: (i, 0)),
            pl.BlockSpec((112, FP), lambda i: (0, 0)),
        ],
        out_shape=[
            jax.ShapeDtypeStruct((NP, FP), jnp.float32),
            jax.ShapeDtypeStruct((NP, FP), jnp.float32),
            jax.ShapeDtypeStruct((112, FP), jnp.float32),
        ],
    )(h, wi, wj, bf, emb_p, encw, encb, we)


def _combine_body(stats_ref, c_ref, h_ref, cnt_ref, cntf_ref,
                  px_ref, ga_ref, gm_ref, gt_ref, bpost_ref, lin_ref, linb_ref,
                  y_ref, psum_ref, psq_ref):
    i = pl.program_id(0)
    cnt_full = cntf_ref[...]
    rows_full = lax.broadcasted_iota(jnp.int32, (NP, 1), 0)
    valid_full = rows_full < N
    avg_log = jnp.sum(jnp.where(valid_full, jnp.log(cnt_full + 1.0), 0.0)) / float(N)

    cnt = cnt_ref[...]
    deg = jnp.maximum(cnt, 1.0)
    logdeg = jnp.log(deg + 1.0)
    sa = logdeg / avg_log
    sb = avg_log / logdeg
    has = cnt > 0.0

    st = stats_ref[...]
    s_sum = st[:, 0:FP]
    s_sq = st[:, FP:2 * FP]
    s_mn = st[:, 2 * FP:3 * FP]
    s_mx = st[:, 3 * FP:4 * FP]
    c = c_ref[...]
    mean_w = s_sum / deg
    mean = jnp.where(has, c + mean_w, 0.0)
    mn = jnp.where(has, c + s_mn, 0.0)
    mx = jnp.where(has, c + s_mx, 0.0)
    var = jax.nn.relu(s_sq / deg - mean_w * mean_w)
    std = jnp.where(has, jnp.sqrt(var + 1e-5), jnp.sqrt(1e-5))
    s_cat = jnp.concatenate([mean, mn, mx, std], axis=-1)

    hb = h_ref[...].astype(BF)
    y75 = (jnp.dot(hb, px_ref[...].astype(BF), preferred_element_type=jnp.float32)
           + jnp.dot(s_cat.astype(BF), ga_ref[...].astype(BF),
                     preferred_element_type=jnp.float32)
           + jnp.dot((sa * s_cat).astype(BF), gm_ref[...].astype(BF),
                     preferred_element_type=jnp.float32)
           + jnp.dot((sb * s_cat).astype(BF), gt_ref[...].astype(BF),
                     preferred_element_type=jnp.float32)
           + bpost_ref[...])
    y = jnp.dot(y75.astype(BF), lin_ref[...].astype(BF),
                preferred_element_type=jnp.float32) + linb_ref[...]
    y_ref[...] = y

    grow = lax.broadcasted_iota(jnp.int32, (BLK, 1), 0) + BLK * i
    vmask = grow < N
    ym_ = jnp.where(vmask, y, 0.0)
    psum_ref[...] = jnp.sum(ym_, axis=0).reshape(1, 1, FH)
    psq_ref[...] = jnp.sum(ym_ * ym_, axis=0).reshape(1, 1, FH)


def _combine(stats, c, h, cnt, px, ga, gm, gt, bpost, lin, linb):
    return pl.pallas_call(
        _combine_body,
        grid=(NB,),
        in_specs=[
            pl.BlockSpec((BLK, 4 * FP), lambda i: (i, 0)),
            pl.BlockSpec((BLK, FP), lambda i: (i, 0)),
            pl.BlockSpec((BLK, FH), lambda i: (i, 0)),
            pl.BlockSpec((BLK, 1), lambda i: (i, 0)),
            pl.BlockSpec((NP, 1), lambda i: (0, 0)),
            pl.BlockSpec((FH, FH), lambda i: (0, 0)),
            pl.BlockSpec((4 * FP, FH), lambda i: (0, 0)),
            pl.BlockSpec((4 * FP, FH), lambda i: (0, 0)),
            pl.BlockSpec((4 * FP, FH), lambda i: (0, 0)),
            pl.BlockSpec((1, FH), lambda i: (0, 0)),
            pl.BlockSpec((FH, FH), lambda i: (0, 0)),
            pl.BlockSpec((1, FH), lambda i: (0, 0)),
        ],
        out_specs=[
            pl.BlockSpec((BLK, FH), lambda i: (i, 0)),
            pl.BlockSpec((1, 1, FH), lambda i: (i, 0, 0)),
            pl.BlockSpec((1, 1, FH), lambda i: (i, 0, 0)),
        ],
        out_shape=[
            jax.ShapeDtypeStruct((NP, FH), jnp.float32),
            jax.ShapeDtypeStruct((NB, 1, FH), jnp.float32),
            jax.ShapeDtypeStruct((NB, 1, FH), jnp.float32),
        ],
    )(stats, c, h, cnt, cnt, px, ga, gm, gt, bpost, lin, linb)


def _bn_body(y_ref, psum_ref, psq_ref, g_ref, b_ref, h_ref):
    i = pl.program_id(0)
    mu = jnp.sum(psum_ref[...], axis=0) / float(N)
    var = jnp.sum(psq_ref[...], axis=0) / float(N) - mu * mu
    inv = 1.0 / jnp.sqrt(var + 1e-5)
    y = (y_ref[...] - mu) * inv * g_ref[...] + b_ref[...]
    grow = lax.broadcasted_iota(jnp.int32, (BLK, 1), 0) + BLK * i
    h_ref[...] = jnp.where(grow < N, jax.nn.relu(y), 0.0)


def _bn(y, psum, psq, g, b):
    return pl.pallas_call(
        _bn_body,
        grid=(NB,),
        in_specs=[
            pl.BlockSpec((BLK, FH), lambda i: (i, 0)),
            pl.BlockSpec((NB, 1, FH), lambda i: (0, 0, 0)),
            pl.BlockSpec((NB, 1, FH), lambda i: (0, 0, 0)),
            pl.BlockSpec((1, FH), lambda i: (0, 0)),
            pl.BlockSpec((1, FH), lambda i: (0, 0)),
        ],
        out_specs=pl.BlockSpec((BLK, FH), lambda i: (i, 0)),
        out_shape=jax.ShapeDtypeStruct((NP, FH), jnp.float32),
    )(y, psum, psq, g, b)


def _pool_body(h_ref, bi_ref, w1_ref, b1_ref, w2_ref, b2_ref, w3_ref, b3_ref,
               acc_ref, out_ref):
    i = pl.program_id(0)

    @pl.when(i == 0)
    def _():
        acc_ref[...] = jnp.zeros_like(acc_ref)

    bi = bi_ref[...].reshape(1, BLK)
    gids = lax.broadcasted_iota(jnp.int32, (N_GRAPHS, BLK), 0)
    onehot_t = (gids == bi).astype(jnp.float32)
    acc_ref[...] += jnp.dot(onehot_t, h_ref[...], preferred_element_type=jnp.float32)

    @pl.when(i == NB - 1)
    def _():
        pooled = acc_ref[...]
        z = jax.nn.relu(jnp.dot(pooled.astype(BF), w1_ref[...].astype(BF),
                                preferred_element_type=jnp.float32) + b1_ref[...])
        z = jax.nn.relu(jnp.dot(z.astype(BF), w2_ref[...].astype(BF),
                                preferred_element_type=jnp.float32) + b2_ref[...])
        out_ref[...] = jnp.dot(z.astype(BF), w3_ref[...].astype(BF),
                               preferred_element_type=jnp.float32) + b3_ref[...]


def _pool_mlp(h, bi, w1, b1, w2, b2, w3, b3):
    acc, out = pl.pallas_call(
        _pool_body,
        grid=(NB,),
        in_specs=[
            pl.BlockSpec((BLK, FH), lambda i: (i, 0)),
            pl.BlockSpec((BLK, 1), lambda i: (i, 0)),
            pl.BlockSpec((FH, 64), lambda i: (0, 0)),
            pl.BlockSpec((1, 64), lambda i: (0, 0)),
            pl.BlockSpec((64, 32), lambda i: (0, 0)),
            pl.BlockSpec((1, 32), lambda i: (0, 0)),
            pl.BlockSpec((32, 128), lambda i: (0, 0)),
            pl.BlockSpec((1, 128), lambda i: (0, 0)),
        ],
        out_specs=[
            pl.BlockSpec((N_GRAPHS, FH), lambda i: (0, 0)),
            pl.BlockSpec((N_GRAPHS, 128), lambda i: (0, 0)),
        ],
        out_shape=[
            jax.ShapeDtypeStruct((N_GRAPHS, FH), jnp.float32),
            jax.ShapeDtypeStruct((N_GRAPHS, 128), jnp.float32),
        ],
    )(h, bi, w1, b1, w2, b2, w3, b3)
    return out


def _prep_layer_weights(l, pre_W, pre_b, post_W, post_b, lin_W, lin_b,
                        edge_enc_W, edge_enc_b):
    Wi = pre_W[l][:, 0:F_IN, :].transpose(1, 0, 2).reshape(F_IN, F3)
    Wj = pre_W[l][:, F_IN:2 * F_IN, :].transpose(1, 0, 2).reshape(F_IN, F3)
    We = pre_W[l][:, 2 * F_IN:, :].transpose(1, 0, 2).reshape(F_IN, F3)
    b_flat = pre_b[l].reshape(F3)

    wi_p = jnp.zeros((FH, FP)).at[:F_IN, :F3].set(Wi)
    wj_p = jnp.zeros((FH, FP)).at[:F_IN, :F3].set(Wj)
    bf_p = jnp.zeros((1, FP)).at[0, :F3].set(b_flat)
    we_p = jnp.zeros((FH, FP)).at[:F_IN, :F3].set(We)

    encw_p = jnp.zeros((64, FH)).at[:50, :F_IN].set(edge_enc_W[l])
    encb_p = jnp.zeros((1, FH)).at[0, :F_IN].set(edge_enc_b[l])

    pw = post_W[l]

    def blockdiag(P):
        z = jnp.zeros((F3, TOWERS * F_OUT))
        for t in range(TOWERS):
            z = z.at[t * F_IN:(t + 1) * F_IN, t * F_OUT:(t + 1) * F_OUT].set(P[t])
        return z

    def gmat(row0):
        gs = []
        for s in range(4):
            bd = blockdiag(pw[:, row0 + s * F_IN: row0 + (s + 1) * F_IN, :])
            gs.append(jnp.zeros((FP, FH)).at[:F3, :TOWERS * F_OUT].set(bd))
        return jnp.concatenate(gs, axis=0)

    PX = jnp.zeros((F_IN, TOWERS * F_OUT))
    for t in range(TOWERS):
        PX = PX.at[:, t * F_OUT:(t + 1) * F_OUT].set(pw[t, 0:F_IN, :])
    px_p = jnp.zeros((FH, FH)).at[:F_IN, :TOWERS * F_OUT].set(PX)

    ga = gmat(F_IN)
    gm = gmat(F_IN + 300)
    gt = gmat(F_IN + 600)

    bpost = jnp.zeros((1, FH)).at[0, :TOWERS * F_OUT].set(post_b[l].reshape(TOWERS * F_OUT))
    lin_p = jnp.zeros((FH, FH)).at[:TOWERS * F_OUT, :F_IN].set(lin_W[l])
    linb_p = jnp.zeros((1, FH)).at[0, :F_IN].set(lin_b[l])
    return (wi_p, wj_p, bf_p, we_p, encw_p, encb_p, px_p, ga, gm, gt,
            bpost, lin_p, linb_p)


def _dbg_my_structure_jnp(x_idx, edge_index, edge_attr_idx, batch_index,
                          node_emb, edge_emb, edge_enc_W, edge_enc_b, pre_W,
                          pre_b, post_W, post_b, lin_W, lin_b, bn_g, bn_b,
                          mlp_W1, mlp_b1, mlp_W2, mlp_b2, mlp_W3, mlp_b3):
    f32 = jnp.float32
    def mm(a, b):
        return jnp.matmul(a.astype(BF), b.astype(BF), preferred_element_type=f32)
    src = edge_index[0]
    dst = edge_index[1]
    h = node_emb[x_idx]
    cnt = jax.ops.segment_sum(jnp.ones((E,), f32), dst, num_segments=N)
    avg_log = jnp.mean(jnp.log(cnt + 1.0))
    deg = jnp.clip(cnt, 1.0, None)
    logdeg = jnp.log(deg + 1.0)
    has = (cnt > 0)[:, None]
    for l in range(L):
        Wi = pre_W[l][:, 0:F_IN, :].transpose(1, 0, 2).reshape(F_IN, F3)
        Wj = pre_W[l][:, F_IN:2 * F_IN, :].transpose(1, 0, 2).reshape(F_IN, F3)
        We = pre_W[l][:, 2 * F_IN:, :].transpose(1, 0, 2).reshape(F_IN, F3)
        b_flat = pre_b[l].reshape(F3)
        C = mm(h, Wi) + b_flat
        Wn = mm(h, Wj)
        ea_tab = mm(edge_emb, edge_enc_W[l]) + edge_enc_b[l]
        T = mm(ea_tab, We)
        w = Wn[src] + T[edge_attr_idx]
        s1 = jax.ops.segment_sum(w, dst, num_segments=N)
        s2 = jax.ops.segment_sum(w * w, dst, num_segments=N)
        mean_w = s1 / deg[:, None]
        mean = jnp.where(has, C + mean_w, 0.0)
        mn = jnp.where(has, C + jax.ops.segment_min(w, dst, num_segments=N), 0.0)
        mx = jnp.where(has, C + jax.ops.segment_max(w, dst, num_segments=N), 0.0)
        var = jax.nn.relu(s2 / deg[:, None] - mean_w * mean_w)
        std = jnp.where(has, jnp.sqrt(var + 1e-5), jnp.sqrt(1e-5))
        s_cat = jnp.concatenate([mean, mn, mx, std], axis=-1)
        pw = post_W[l]
        def blockdiag(P):
            z = jnp.zeros((F3, TOWERS * F_OUT))
            for t in range(TOWERS):
                z = z.at[t * F_IN:(t + 1) * F_IN, t * F_OUT:(t + 1) * F_OUT].set(P[t])
            return z
        def gmat(row0):
            return jnp.concatenate(
                [blockdiag(pw[:, row0 + s * F_IN: row0 + (s + 1) * F_IN, :])
                 for s in range(4)], axis=0)
        PX = jnp.zeros((F_IN, TOWERS * F_OUT))
        for t in range(TOWERS):
            PX = PX.at[:, t * F_OUT:(t + 1) * F_OUT].set(pw[t, 0:F_IN, :])
        ga = gmat(F_IN)
        gm = gmat(F_IN + 300)
        gt = gmat(F_IN + 600)
        sa = (logdeg / avg_log)[:, None]
        sb = (avg_log / logdeg)[:, None]
        y75 = (mm(h, PX) + mm(s_cat, ga) + mm(sa * s_cat, gm)
               + mm(sb * s_cat, gt) + post_b[l].reshape(TOWERS * F_OUT))
        out = mm(y75, lin_W[l]) + lin_b[l]
        mu = jnp.mean(out, axis=0)
        v = jnp.var(out, axis=0)
        out = (out - mu) / jnp.sqrt(v + 1e-5) * bn_g[l] + bn_b[l]
        h = jax.nn.relu(out)
    pooled = jax.ops.segment_sum(h, batch_index, num_segments=N_GRAPHS)
    z = jax.nn.relu(mm(pooled, mlp_W1) + mlp_b1)
    z = jax.nn.relu(mm(z, mlp_W2) + mlp_b2)
    return mm(z, mlp_W3) + mlp_b3


def kernel(x_idx, edge_index, edge_attr_idx, batch_index, node_emb, edge_emb,
           edge_enc_W, edge_enc_b, pre_W, pre_b, post_W, post_b, lin_W, lin_b,
           bn_g, bn_b, mlp_W1, mlp_b1, mlp_W2, mlp_b2, mlp_W3, mlp_b3):
    src = edge_index[0]
    dst = edge_index[1]

    order = jnp.argsort(dst)
    src_s = src[order].astype(jnp.int32)
    ea_s = edge_attr_idx[order].astype(jnp.int32)
    dst_s = dst[order].astype(jnp.int32)
    offsets = jnp.searchsorted(dst_s, jnp.arange(NP + 1, dtype=jnp.int32)).astype(jnp.int32)
    cnt = (offsets[1:] - offsets[:-1]).astype(jnp.float32).reshape(NP, 1)

    srcs_p = jnp.pad(src_s, (0, 2 * CH))
    eas_p = jnp.pad(ea_s, (0, 2 * CH))
    dsts_p = jnp.pad(dst_s, (0, 2 * CH), constant_values=1 << 20)
    bidx = jnp.minimum(
        (jnp.arange(SC_NW)[:, None] * NPW + jnp.arange(32)[None, :] * 16), NP)
    offb = offsets[bidx].reshape(SC_NW * 32)

    h = jnp.zeros((NP, FH), jnp.float32).at[:N, :F_IN].set(node_emb[x_idx])
    bi = jnp.full((NP, 1), -1, jnp.int32).at[:N, 0].set(batch_index.astype(jnp.int32))
    emb_p = jnp.zeros((112, 64), jnp.float32).at[:100, :50].set(edge_emb)

    for l in range(L):
        (wi_p, wj_p, bf_p, we_p, encw_p, encb_p, px_p, ga, gm, gt,
         bpost, lin_p, linb_p) = _prep_layer_weights(
            l, pre_W, pre_b, post_W, post_b, lin_W, lin_b, edge_enc_W, edge_enc_b)
        c_arr, wn_arr, t_arr = _proj(h, wi_p, wj_p, bf_p, emb_p, encw_p,
                                     encb_p, we_p)

        stats = _sc_stats(wn_arr, t_arr, srcs_p, eas_p, dsts_p, offb)
        stats = stats.reshape(NP, 4 * FP)

        y, psum, psq = _combine(stats, c_arr, h, cnt, px_p, ga, gm, gt,
                                bpost, lin_p, linb_p)
        g_p = jnp.zeros((1, FH)).at[0, :F_IN].set(bn_g[l])
        b_p = jnp.zeros((1, FH)).at[0, :F_IN].set(bn_b[l])
        h = _bn(y, psum, psq, g_p, b_p)

    w1_p = jnp.zeros((FH, 64)).at[:F_IN, :50].set(mlp_W1)
    b1_p = jnp.zeros((1, 64)).at[0, :50].set(mlp_b1)
    w2_p = jnp.zeros((64, 32)).at[:50, :25].set(mlp_W2)
    b2_p = jnp.zeros((1, 32)).at[0, :25].set(mlp_b2)
    w3_p = jnp.zeros((32, 128)).at[:25, :1].set(mlp_W3)
    b3_p = jnp.zeros((1, 128)).at[0, :1].set(mlp_b3)
    out = _pool_mlp(h, bi, w1_p, b1_p, w2_p, b2_p, w3_p, b3_p)
    return out[:, :1]


# speedup vs baseline: 14.3919x; 1.0066x over previous
import functools

import jax
import jax.numpy as jnp
from jax import lax
from jax.experimental import pallas as pl
from jax.experimental.pallas import tpu as pltpu
from jax.experimental.pallas import tpu_sc as plsc

N = 10000
E = 160000
L = 2
TOWERS = 5
F_IN = 75
F_OUT = 15
N_GRAPHS = 64

NP = 10240
FP = 384
FH = 128
BLK = 1280
NB = NP // BLK
BIG = 1e30
BF = jnp.bfloat16
F3 = TOWERS * F_IN


SC_NC = 2
SC_NS = 16
SC_NW = SC_NC * SC_NS
NPW = NP // SC_NW
NBLK = NPW // 16
CH = 64
KV = FP // 16
EPAD = E + 2 * CH


def _sload(ref, i):
    return ref[pl.ds(i, 16)][0]


def _sc_stats_body(wn_hbm, t_hbm, srcs_hbm, eas_hbm, dsts_hbm, offb_hbm,
                   out_hbm, srcbuf, eabuf, dstv, offv, wrows, tall, stage,
                   sem1):
    wid = lax.axis_index("s") * SC_NC + lax.axis_index("c")
    n0 = wid * NPW

    pltpu.sync_copy(offb_hbm.at[pl.ds(wid * 32, 32)], offv.at[pl.ds(0, 32)])
    pltpu.sync_copy(t_hbm, tall)

    def block_body(blk, _):
        sl = lax.rem(blk, 2)
        blk_n0 = n0 + blk * 16
        e0b = _sload(offv, blk)
        e1b = _sload(offv, blk + 1)
        c0 = pl.multiple_of(e0b & ~7, 8)
        nch = lax.div(e1b - c0 + (CH - 1), CH)

        zed = jnp.zeros((16,), jnp.float32)
        big = jnp.full((16,), BIG, jnp.float32)

        def init_body(s, _):
            for k in range(KV):
                d = pl.ds(k * 16, 16)
                stage[sl, s, 0, d] = zed
                stage[sl, s, 1, d] = zed
                stage[sl, s, 2, d] = big
                stage[sl, s, 3, d] = -big
            return 0

        lax.fori_loop(0, 16, init_body, 0)

        def chunk_body(ci, _):
            ce = pl.multiple_of(c0 + ci * CH, 8)
            pltpu.sync_copy(srcs_hbm.at[pl.ds(ce, CH)], srcbuf)
            pltpu.sync_copy(eas_hbm.at[pl.ds(ce, CH)], eabuf.at[pl.ds(0, CH)])
            pltpu.sync_copy(dsts_hbm.at[pl.ds(ce, CH)], dstv.at[pl.ds(0, CH)])
            g1 = pltpu.make_async_copy(wn_hbm.at[srcbuf], wrows, sem1)
            g1.start()
            g1.wait()

            def edge_body(j, _):
                slot = _sload(dstv, j) - blk_n0

                @pl.when(jnp.logical_and(slot >= 0, slot < 16))
                def _():
                    ea_j = _sload(eabuf, j)
                    for k in range(KV):
                        d = pl.ds(k * 16, 16)
                        w = wrows[j, d] + tall[ea_j, d]
                        stage[sl, slot, 0, d] += w
                        stage[sl, slot, 1, d] += w * w
                        stage[sl, slot, 2, d] = jnp.minimum(stage[sl, slot, 2, d], w)
                        stage[sl, slot, 3, d] = jnp.maximum(stage[sl, slot, 3, d], w)

                return 0

            lax.fori_loop(0, CH, edge_body, 0)
            return 0

        lax.fori_loop(0, nch, chunk_body, 0)
        pltpu.sync_copy(stage.at[sl], out_hbm.at[pl.ds(blk_n0, 16)])
        return 0

    lax.fori_loop(0, NBLK, block_body, 0)


def _sc_stats(wn, t, srcs, eas, dsts, offb):
    f = functools.partial(
        pl.kernel,
        mesh=plsc.VectorSubcoreMesh(core_axis_name="c", subcore_axis_name="s"),
        out_type=jax.ShapeDtypeStruct((NP, 4, FP), jnp.float32),
        scratch_types=[
            pltpu.VMEM((CH,), jnp.int32),
            pltpu.VMEM((CH + 16,), jnp.int32),
            pltpu.VMEM((CH + 16,), jnp.int32),
            pltpu.VMEM((48,), jnp.int32),
            pltpu.VMEM((CH, FP), jnp.float32),
            pltpu.VMEM((112, FP), jnp.float32),
            pltpu.VMEM((2, 16, 4, FP), jnp.float32),
            pltpu.SemaphoreType.DMA,
        ],
    )(_sc_stats_body)
    return f(wn, t, srcs, eas, dsts, offb)


def _proj_body(h_ref, wi_ref, wj_ref, bf_ref, emb_ref, encw_ref, encb_ref,
               we_ref, c_ref, wn_ref, t_ref):
    hb = h_ref[...].astype(BF)
    c_ref[...] = jnp.dot(hb, wi_ref[...].astype(BF),
                         preferred_element_type=jnp.float32) + bf_ref[...]
    wn_ref[...] = jnp.dot(hb, wj_ref[...].astype(BF),
                          preferred_element_type=jnp.float32)
    ea = jnp.dot(emb_ref[...].astype(BF), encw_ref[...].astype(BF),
                 preferred_element_type=jnp.float32) + encb_ref[...]
    t_ref[...] = jnp.dot(ea.astype(BF), we_ref[...].astype(BF),
                         preferred_element_type=jnp.float32)


def _proj(h, wi, wj, bf, emb_p, encw, encb, we):
    return pl.pallas_call(
        _proj_body,
        grid=(NB,),
        in_specs=[
            pl.BlockSpec((BLK, FH), lambda i: (i, 0)),
            pl.BlockSpec((FH, FP), lambda i: (0, 0)),
            pl.BlockSpec((FH, FP), lambda i: (0, 0)),
            pl.BlockSpec((1, FP), lambda i: (0, 0)),
            pl.BlockSpec((112, 64), lambda i: (0, 0)),
            pl.BlockSpec((64, FH), lambda i: (0, 0)),
            pl.BlockSpec((1, FH), lambda i: (0, 0)),
            pl.BlockSpec((FH, FP), lambda i: (0, 0)),
        ],
        out_specs=[
            pl.BlockSpec((BLK, FP), lambda i: (i, 0)),
            pl.BlockSpec((BLK, FP), lambda i: (i, 0)),
            pl.BlockSpec((112, FP), lambda i: (0, 0)),
        ],
        out_shape=[
            jax.ShapeDtypeStruct((NP, FP), jnp.float32),
            jax.ShapeDtypeStruct((NP, FP), jnp.float32),
            jax.ShapeDtypeStruct((112, FP), jnp.float32),
        ],
    )(h, wi, wj, bf, emb_p, encw, encb, we)


def _combine_body(stats_ref, c_ref, h_ref, cnt_ref, cntf_ref,
                  px_ref, ga_ref, gm_ref, gt_ref, bpost_ref, lin_ref, linb_ref,
                  y_ref, psum_ref, psq_ref):
    i = pl.program_id(0)
    cnt_full = cntf_ref[...]
    rows_full = lax.broadcasted_iota(jnp.int32, (NP, 1), 0)
    valid_full = rows_full < N
    avg_log = jnp.sum(jnp.where(valid_full, jnp.log(cnt_full + 1.0), 0.0)) / float(N)

    cnt = cnt_ref[...]
    deg = jnp.maximum(cnt, 1.0)
    logdeg = jnp.log(deg + 1.0)
    sa = logdeg / avg_log
    sb = avg_log / logdeg
    has = cnt > 0.0

    st = stats_ref[...]
    s_sum = st[:, 0:FP]
    s_sq = st[:, FP:2 * FP]
    s_mn = st[:, 2 * FP:3 * FP]
    s_mx = st[:, 3 * FP:4 * FP]
    c = c_ref[...]
    mean_w = s_sum / deg
    mean = jnp.where(has, c + mean_w, 0.0)
    mn = jnp.where(has, c + s_mn, 0.0)
    mx = jnp.where(has, c + s_mx, 0.0)
    var = jax.nn.relu(s_sq / deg - mean_w * mean_w)
    std = jnp.where(has, jnp.sqrt(var + 1e-5), jnp.sqrt(1e-5))
    s_cat = jnp.concatenate([mean, mn, mx, std], axis=-1)

    hb = h_ref[...].astype(BF)
    y75 = (jnp.dot(hb, px_ref[...].astype(BF), preferred_element_type=jnp.float32)
           + jnp.dot(s_cat.astype(BF), ga_ref[...].astype(BF),
                     preferred_element_type=jnp.float32)
           + jnp.dot((sa * s_cat).astype(BF), gm_ref[...].astype(BF),
                     preferred_element_type=jnp.float32)
           + jnp.dot((sb * s_cat).astype(BF), gt_ref[...].astype(BF),
                     preferred_element_type=jnp.float32)
           + bpost_ref[...])
    y = jnp.dot(y75.astype(BF), lin_ref[...].astype(BF),
                preferred_element_type=jnp.float32) + linb_ref[...]
    y_ref[...] = y

    grow = lax.broadcasted_iota(jnp.int32, (BLK, 1), 0) + BLK * i
    vmask = grow < N
    ym_ = jnp.where(vmask, y, 0.0)
    psum_ref[...] = jnp.sum(ym_, axis=0).reshape(1, 1, FH)
    psq_ref[...] = jnp.sum(ym_ * ym_, axis=0).reshape(1, 1, FH)


def _combine(stats, c, h, cnt, px, ga, gm, gt, bpost, lin, linb):
    return pl.pallas_call(
        _combine_body,
        grid=(NB,),
        in_specs=[
            pl.BlockSpec((BLK, 4 * FP), lambda i: (i, 0)),
            pl.BlockSpec((BLK, FP), lambda i: (i, 0)),
            pl.BlockSpec((BLK, FH), lambda i: (i, 0)),
            pl.BlockSpec((BLK, 1), lambda i: (i, 0)),
            pl.BlockSpec((NP, 1), lambda i: (0, 0)),
            pl.BlockSpec((FH, FH), lambda i: (0, 0)),
            pl.BlockSpec((4 * FP, FH), lambda i: (0, 0)),
            pl.BlockSpec((4 * FP, FH), lambda i: (0, 0)),
            pl.BlockSpec((4 * FP, FH), lambda i: (0, 0)),
            pl.BlockSpec((1, FH), lambda i: (0, 0)),
            pl.BlockSpec((FH, FH), lambda i: (0, 0)),
            pl.BlockSpec((1, FH), lambda i: (0, 0)),
        ],
        out_specs=[
            pl.BlockSpec((BLK, FH), lambda i: (i, 0)),
            pl.BlockSpec((1, 1, FH), lambda i: (i, 0, 0)),
            pl.BlockSpec((1, 1, FH), lambda i: (i, 0, 0)),
        ],
        out_shape=[
            jax.ShapeDtypeStruct((NP, FH), jnp.float32),
            jax.ShapeDtypeStruct((NB, 1, FH), jnp.float32),
            jax.ShapeDtypeStruct((NB, 1, FH), jnp.float32),
        ],
    )(stats, c, h, cnt, cnt, px, ga, gm, gt, bpost, lin, linb)


def _bn_body(y_ref, psum_ref, psq_ref, g_ref, b_ref, h_ref):
    i = pl.program_id(0)
    mu = jnp.sum(psum_ref[...], axis=0) / float(N)
    var = jnp.sum(psq_ref[...], axis=0) / float(N) - mu * mu
    inv = 1.0 / jnp.sqrt(var + 1e-5)
    y = (y_ref[...] - mu) * inv * g_ref[...] + b_ref[...]
    grow = lax.broadcasted_iota(jnp.int32, (BLK, 1), 0) + BLK * i
    h_ref[...] = jnp.where(grow < N, jax.nn.relu(y), 0.0)


def _bn(y, psum, psq, g, b):
    return pl.pallas_call(
        _bn_body,
        grid=(NB,),
        in_specs=[
            pl.BlockSpec((BLK, FH), lambda i: (i, 0)),
            pl.BlockSpec((NB, 1, FH), lambda i: (0, 0, 0)),
            pl.BlockSpec((NB, 1, FH), lambda i: (0, 0, 0)),
            pl.BlockSpec((1, FH), lambda i: (0, 0)),
            pl.BlockSpec((1, FH), lambda i: (0, 0)),
        ],
        out_specs=pl.BlockSpec((BLK, FH), lambda i: (i, 0)),
        out_shape=jax.ShapeDtypeStruct((NP, FH), jnp.float32),
    )(y, psum, psq, g, b)


def _pool_body(h_ref, bi_ref, w1_ref, b1_ref, w2_ref, b2_ref, w3_ref, b3_ref,
               acc_ref, out_ref):
    i = pl.program_id(0)

    @pl.when(i == 0)
    def _():
        acc_ref[...] = jnp.zeros_like(acc_ref)

    bi = bi_ref[...].reshape(1, BLK)
    gids = lax.broadcasted_iota(jnp.int32, (N_GRAPHS, BLK), 0)
    onehot_t = (gids == bi).astype(jnp.float32)
    acc_ref[...] += jnp.dot(onehot_t, h_ref[...], preferred_element_type=jnp.float32)

    @pl.when(i == NB - 1)
    def _():
        pooled = acc_ref[...]
        z = jax.nn.relu(jnp.dot(pooled.astype(BF), w1_ref[...].astype(BF),
                                preferred_element_type=jnp.float32) + b1_ref[...])
        z = jax.nn.relu(jnp.dot(z.astype(BF), w2_ref[...].astype(BF),
                                preferred_element_type=jnp.float32) + b2_ref[...])
        out_ref[...] = jnp.dot(z.astype(BF), w3_ref[...].astype(BF),
                               preferred_element_type=jnp.float32) + b3_ref[...]


def _pool_mlp(h, bi, w1, b1, w2, b2, w3, b3):
    acc, out = pl.pallas_call(
        _pool_body,
        grid=(NB,),
        in_specs=[
            pl.BlockSpec((BLK, FH), lambda i: (i, 0)),
            pl.BlockSpec((BLK, 1), lambda i: (i, 0)),
            pl.BlockSpec((FH, 64), lambda i: (0, 0)),
            pl.BlockSpec((1, 64), lambda i: (0, 0)),
            pl.BlockSpec((64, 32), lambda i: (0, 0)),
            pl.BlockSpec((1, 32), lambda i: (0, 0)),
            pl.BlockSpec((32, 128), lambda i: (0, 0)),
            pl.BlockSpec((1, 128), lambda i: (0, 0)),
        ],
        out_specs=[
            pl.BlockSpec((N_GRAPHS, FH), lambda i: (0, 0)),
            pl.BlockSpec((N_GRAPHS, 128), lambda i: (0, 0)),
        ],
        out_shape=[
            jax.ShapeDtypeStruct((N_GRAPHS, FH), jnp.float32),
            jax.ShapeDtypeStruct((N_GRAPHS, 128), jnp.float32),
        ],
    )(h, bi, w1, b1, w2, b2, w3, b3)
    return out


def _prep_layer_weights(l, pre_W, pre_b, post_W, post_b, lin_W, lin_b,
                        edge_enc_W, edge_enc_b):
    Wi = pre_W[l][:, 0:F_IN, :].transpose(1, 0, 2).reshape(F_IN, F3)
    Wj = pre_W[l][:, F_IN:2 * F_IN, :].transpose(1, 0, 2).reshape(F_IN, F3)
    We = pre_W[l][:, 2 * F_IN:, :].transpose(1, 0, 2).reshape(F_IN, F3)
    b_flat = pre_b[l].reshape(F3)

    wi_p = jnp.zeros((FH, FP)).at[:F_IN, :F3].set(Wi)
    wj_p = jnp.zeros((FH, FP)).at[:F_IN, :F3].set(Wj)
    bf_p = jnp.zeros((1, FP)).at[0, :F3].set(b_flat)
    we_p = jnp.zeros((FH, FP)).at[:F_IN, :F3].set(We)

    encw_p = jnp.zeros((64, FH)).at[:50, :F_IN].set(edge_enc_W[l])
    encb_p = jnp.zeros((1, FH)).at[0, :F_IN].set(edge_enc_b[l])

    pw = post_W[l]

    def blockdiag(P):
        z = jnp.zeros((F3, TOWERS * F_OUT))
        for t in range(TOWERS):
            z = z.at[t * F_IN:(t + 1) * F_IN, t * F_OUT:(t + 1) * F_OUT].set(P[t])
        return z

    def gmat(row0):
        gs = []
        for s in range(4):
            bd = blockdiag(pw[:, row0 + s * F_IN: row0 + (s + 1) * F_IN, :])
            gs.append(jnp.zeros((FP, FH)).at[:F3, :TOWERS * F_OUT].set(bd))
        return jnp.concatenate(gs, axis=0)

    PX = jnp.zeros((F_IN, TOWERS * F_OUT))
    for t in range(TOWERS):
        PX = PX.at[:, t * F_OUT:(t + 1) * F_OUT].set(pw[t, 0:F_IN, :])
    px_p = jnp.zeros((FH, FH)).at[:F_IN, :TOWERS * F_OUT].set(PX)

    ga = gmat(F_IN)
    gm = gmat(F_IN + 300)
    gt = gmat(F_IN + 600)

    bpost = jnp.zeros((1, FH)).at[0, :TOWERS * F_OUT].set(post_b[l].reshape(TOWERS * F_OUT))
    lin_p = jnp.zeros((FH, FH)).at[:TOWERS * F_OUT, :F_IN].set(lin_W[l])
    linb_p = jnp.zeros((1, FH)).at[0, :F_IN].set(lin_b[l])
    return (wi_p, wj_p, bf_p, we_p, encw_p, encb_p, px_p, ga, gm, gt,
            bpost, lin_p, linb_p)


def _dbg_my_structure_jnp(x_idx, edge_index, edge_attr_idx, batch_index,
                          node_emb, edge_emb, edge_enc_W, edge_enc_b, pre_W,
                          pre_b, post_W, post_b, lin_W, lin_b, bn_g, bn_b,
                          mlp_W1, mlp_b1, mlp_W2, mlp_b2, mlp_W3, mlp_b3):
    f32 = jnp.float32
    def mm(a, b):
        return jnp.matmul(a.astype(BF), b.astype(BF), preferred_element_type=f32)
    src = edge_index[0]
    dst = edge_index[1]
    h = node_emb[x_idx]
    cnt = jax.ops.segment_sum(jnp.ones((E,), f32), dst, num_segments=N)
    avg_log = jnp.mean(jnp.log(cnt + 1.0))
    deg = jnp.clip(cnt, 1.0, None)
    logdeg = jnp.log(deg + 1.0)
    has = (cnt > 0)[:, None]
    for l in range(L):
        Wi = pre_W[l][:, 0:F_IN, :].transpose(1, 0, 2).reshape(F_IN, F3)
        Wj = pre_W[l][:, F_IN:2 * F_IN, :].transpose(1, 0, 2).reshape(F_IN, F3)
        We = pre_W[l][:, 2 * F_IN:, :].transpose(1, 0, 2).reshape(F_IN, F3)
        b_flat = pre_b[l].reshape(F3)
        C = mm(h, Wi) + b_flat
        Wn = mm(h, Wj)
        ea_tab = mm(edge_emb, edge_enc_W[l]) + edge_enc_b[l]
        T = mm(ea_tab, We)
        w = Wn[src] + T[edge_attr_idx]
        s1 = jax.ops.segment_sum(w, dst, num_segments=N)
        s2 = jax.ops.segment_sum(w * w, dst, num_segments=N)
        mean_w = s1 / deg[:, None]
        mean = jnp.where(has, C + mean_w, 0.0)
        mn = jnp.where(has, C + jax.ops.segment_min(w, dst, num_segments=N), 0.0)
        mx = jnp.where(has, C + jax.ops.segment_max(w, dst, num_segments=N), 0.0)
        var = jax.nn.relu(s2 / deg[:, None] - mean_w * mean_w)
        std = jnp.where(has, jnp.sqrt(var + 1e-5), jnp.sqrt(1e-5))
        s_cat = jnp.concatenate([mean, mn, mx, std], axis=-1)
        pw = post_W[l]
        def blockdiag(P):
            z = jnp.zeros((F3, TOWERS * F_OUT))
            for t in range(TOWERS):
                z = z.at[t * F_IN:(t + 1) * F_IN, t * F_OUT:(t + 1) * F_OUT].set(P[t])
            return z
        def gmat(row0):
            return jnp.concatenate(
                [blockdiag(pw[:, row0 + s * F_IN: row0 + (s + 1) * F_IN, :])
                 for s in range(4)], axis=0)
        PX = jnp.zeros((F_IN, TOWERS * F_OUT))
        for t in range(TOWERS):
            PX = PX.at[:, t * F_OUT:(t + 1) * F_OUT].set(pw[t, 0:F_IN, :])
        ga = gmat(F_IN)
        gm = gmat(F_IN + 300)
        gt = gmat(F_IN + 600)
        sa = (logdeg / avg_log)[:, None]
        sb = (avg_log / logdeg)[:, None]
        y75 = (mm(h, PX) + mm(s_cat, ga) + mm(sa * s_cat, gm)
               + mm(sb * s_cat, gt) + post_b[l].reshape(TOWERS * F_OUT))
        out = mm(y75, lin_W[l]) + lin_b[l]
        mu = jnp.mean(out, axis=0)
        v = jnp.var(out, axis=0)
        out = (out - mu) / jnp.sqrt(v + 1e-5) * bn_g[l] + bn_b[l]
        h = jax.nn.relu(out)
    pooled = jax.ops.segment_sum(h, batch_index, num_segments=N_GRAPHS)
    z = jax.nn.relu(mm(pooled, mlp_W1) + mlp_b1)
    z = jax.nn.relu(mm(z, mlp_W2) + mlp_b2)
    return mm(z, mlp_W3) + mlp_b3


def kernel(x_idx, edge_index, edge_attr_idx, batch_index, node_emb, edge_emb,
           edge_enc_W, edge_enc_b, pre_W, pre_b, post_W, post_b, lin_W, lin_b,
           bn_g, bn_b, mlp_W1, mlp_b1, mlp_W2, mlp_b2, mlp_W3, mlp_b3):
    src = edge_index[0]
    dst = edge_index[1]

    order = jnp.argsort(dst)
    src_s = src[order].astype(jnp.int32)
    ea_s = edge_attr_idx[order].astype(jnp.int32)
    dst_s = dst[order].astype(jnp.int32)
    offsets = jnp.searchsorted(dst_s, jnp.arange(NP + 1, dtype=jnp.int32)).astype(jnp.int32)
    cnt = (offsets[1:] - offsets[:-1]).astype(jnp.float32).reshape(NP, 1)

    srcs_p = jnp.pad(src_s, (0, 2 * CH))
    eas_p = jnp.pad(ea_s, (0, 2 * CH))
    dsts_p = jnp.pad(dst_s, (0, 2 * CH), constant_values=1 << 20)
    bidx = jnp.minimum(
        (jnp.arange(SC_NW)[:, None] * NPW + jnp.arange(32)[None, :] * 16), NP)
    offb = offsets[bidx].reshape(SC_NW * 32)

    h = jnp.zeros((NP, FH), jnp.float32).at[:N, :F_IN].set(node_emb[x_idx])
    bi = jnp.full((NP, 1), -1, jnp.int32).at[:N, 0].set(batch_index.astype(jnp.int32))
    emb_p = jnp.zeros((112, 64), jnp.float32).at[:100, :50].set(edge_emb)

    for l in range(L):
        (wi_p, wj_p, bf_p, we_p, encw_p, encb_p, px_p, ga, gm, gt,
         bpost, lin_p, linb_p) = _prep_layer_weights(
            l, pre_W, pre_b, post_W, post_b, lin_W, lin_b, edge_enc_W, edge_enc_b)
        c_arr, wn_arr, t_arr = _proj(h, wi_p, wj_p, bf_p, emb_p, encw_p,
                                     encb_p, we_p)

        stats = _sc_stats(wn_arr, t_arr, srcs_p, eas_p, dsts_p, offb)
        stats = stats.reshape(NP, 4 * FP)

        y, psum, psq = _combine(stats, c_arr, h, cnt, px_p, ga, gm, gt,
                                bpost, lin_p, linb_p)
        g_p = jnp.zeros((1, FH)).at[0, :F_IN].set(bn_g[l])
        b_p = jnp.zeros((1, FH)).at[0, :F_IN].set(bn_b[l])
        h = _bn(y, psum, psq, g_p, b_p)

    w1_p = jnp.zeros((FH, 64)).at[:F_IN, :50].set(mlp_W1)
    b1_p = jnp.zeros((1, 64)).at[0, :50].set(mlp_b1)
    w2_p = jnp.zeros((64, 32)).at[:50, :25].set(mlp_W2)
    b2_p = jnp.zeros((1, 32)).at[0, :25].set(mlp_b2)
    w3_p = jnp.zeros((32, 128)).at[:25, :1].set(mlp_W3)
    b3_p = jnp.zeros((1, 128)).at[0, :1].set(mlp_b3)
    out = _pool_mlp(h, bi, w1_p, b1_p, w2_p, b2_p, w3_p, b3_p)
    return out[:, :1]
